# baseline (speedup 1.0000x reference)
import functools

import jax
import jax.numpy as jnp
from jax import lax
from jax.experimental import pallas as pl
from jax.experimental.pallas import tpu as pltpu
from jax.experimental.pallas import tpu_sc as plsc

_D = 128
_B = 5000
_NC = 2
_NS = 16
_NW = _NC * _NS
_ROWS = 160
_CHUNK = 40
_NCHUNK = _ROWS // _CHUNK
_LAST_BASE = _B - _ROWS

_mesh = plsc.VectorSubcoreMesh(core_axis_name="c", subcore_axis_name="s")


@functools.partial(
    pl.kernel,
    mesh=_mesh,
    out_type=jax.ShapeDtypeStruct((_B, _D), jnp.float32),
    scratch_types=[
        pltpu.VMEM((_ROWS,), jnp.int32),
        pltpu.VMEM((_ROWS, _D), jnp.float32),
    ]
    + [pltpu.SemaphoreType.DMA] * (2 * _NCHUNK)
    + [pltpu.SemaphoreType.DMA],
)
def _gather_rows(idx_hbm, x_hbm, out_hbm, idx_v, rows_v, *sems):
    idx_sems = sems[:_NCHUNK]
    gather_sems = sems[_NCHUNK:2 * _NCHUNK]
    out_sem = sems[2 * _NCHUNK]
    wid = lax.axis_index("s") * _NC + lax.axis_index("c")
    base = pl.multiple_of(lax.min(wid * _ROWS, _LAST_BASE), 8)
    idx_copies = [
        pltpu.async_copy(
            idx_hbm.at[pl.ds(base + j * _CHUNK, _CHUNK)],
            idx_v.at[pl.ds(j * _CHUNK, _CHUNK)],
            idx_sems[j],
        )
        for j in range(_NCHUNK)
    ]
    gathers = []
    for j in range(_NCHUNK):
        idx_copies[j].wait()
        gathers.append(
            pltpu.async_copy(
                x_hbm.at[idx_v.at[pl.ds(j * _CHUNK, _CHUNK)]],
                rows_v.at[pl.ds(j * _CHUNK, _CHUNK)],
                gather_sems[j],
            )
        )
    outs = []
    for j in range(_NCHUNK):
        gathers[j].wait()
        outs.append(
            pltpu.async_copy(
                rows_v.at[pl.ds(j * _CHUNK, _CHUNK)],
                out_hbm.at[pl.ds(base + j * _CHUNK, _CHUNK)],
                out_sem,
            )
        )
    for cp in outs:
        cp.wait()


def kernel(node_indices, x, edge_index, edge_type, edge_attr):
    del edge_index, edge_type, edge_attr
    return _gather_rows(node_indices, x)


# speedup vs baseline: 1.0037x; 1.0037x over previous
import functools

import jax
import jax.numpy as jnp
from jax import lax
from jax.experimental import pallas as pl
from jax.experimental.pallas import tpu as pltpu
from jax.experimental.pallas import tpu_sc as plsc

_D = 128
_B = 5000
_NC = 2
_NS = 16
_NW = _NC * _NS
_ROWS = 160
_CHUNKS = ((0, 8), (8, 24), (32, 64), (96, 64))
_NCHUNK = len(_CHUNKS)
_LAST_BASE = _B - _ROWS

_mesh = plsc.VectorSubcoreMesh(core_axis_name="c", subcore_axis_name="s")


@functools.partial(
    pl.kernel,
    mesh=_mesh,
    out_type=jax.ShapeDtypeStruct((_B, _D), jnp.float32),
    scratch_types=[
        pltpu.VMEM((_ROWS,), jnp.int32),
        pltpu.VMEM((_ROWS, _D), jnp.float32),
    ]
    + [pltpu.SemaphoreType.DMA] * (2 * _NCHUNK)
    + [pltpu.SemaphoreType.DMA],
)
def _gather_rows(idx_hbm, x_hbm, out_hbm, idx_v, rows_v, *sems):
    idx_sems = sems[:_NCHUNK]
    gather_sems = sems[_NCHUNK:2 * _NCHUNK]
    out_sem = sems[2 * _NCHUNK]
    wid = lax.axis_index("s") * _NC + lax.axis_index("c")
    base = pl.multiple_of(lax.min(wid * _ROWS, _LAST_BASE), 8)
    idx_copies = [
        pltpu.async_copy(
            idx_hbm.at[pl.ds(base + off, size)],
            idx_v.at[pl.ds(off, size)],
            idx_sems[j],
        )
        for j, (off, size) in enumerate(_CHUNKS)
    ]
    gathers = []
    for j, (off, size) in enumerate(_CHUNKS):
        idx_copies[j].wait()
        gathers.append(
            pltpu.async_copy(
                x_hbm.at[idx_v.at[pl.ds(off, size)]],
                rows_v.at[pl.ds(off, size)],
                gather_sems[j],
            )
        )
    outs = []
    for j, (off, size) in enumerate(_CHUNKS):
        gathers[j].wait()
        outs.append(
            pltpu.async_copy(
                rows_v.at[pl.ds(off, size)],
                out_hbm.at[pl.ds(base + off, size)],
                out_sem,
            )
        )
    for cp in outs:
        cp.wait()


def kernel(node_indices, x, edge_index, edge_type, edge_attr):
    del edge_index, edge_type, edge_attr
    return _gather_rows(node_indices, x)


# speedup vs baseline: 1.0074x; 1.0037x over previous
import functools

import jax
import jax.numpy as jnp
from jax import lax
from jax.experimental import pallas as pl
from jax.experimental.pallas import tpu as pltpu
from jax.experimental.pallas import tpu_sc as plsc

_D = 128
_B = 5000
_NC = 2
_NS = 16
_NW = _NC * _NS
_ROWS = 160
_CHUNK = 40
_CHUNKS = tuple((j * _CHUNK, _CHUNK) for j in range(_ROWS // _CHUNK))
_NCHUNK = len(_CHUNKS)
_LAST_BASE = _B - _ROWS

_mesh = plsc.VectorSubcoreMesh(core_axis_name="c", subcore_axis_name="s")


@functools.partial(
    pl.kernel,
    mesh=_mesh,
    out_type=jax.ShapeDtypeStruct((_B, _D), jnp.float32),
    scratch_types=[
        pltpu.VMEM((_ROWS,), jnp.int32),
        pltpu.VMEM((_ROWS, _D), jnp.float32),
    ]
    + [pltpu.SemaphoreType.DMA] * (2 * _NCHUNK)
    + [pltpu.SemaphoreType.DMA],
)
def _gather_rows(idx_hbm, x_hbm, out_hbm, idx_v, rows_v, *sems):
    idx_sems = sems[:_NCHUNK]
    gather_sems = sems[_NCHUNK:2 * _NCHUNK]
    out_sem = sems[2 * _NCHUNK]
    wid = lax.axis_index("s") * _NC + lax.axis_index("c")
    base = pl.multiple_of(lax.min(wid * _ROWS, _LAST_BASE), 8)
    idx_copies = [
        pltpu.async_copy(
            idx_hbm.at[pl.ds(base + off, size)],
            idx_v.at[pl.ds(off, size)],
            idx_sems[j],
        )
        for j, (off, size) in enumerate(_CHUNKS)
    ]
    gathers = []
    for j, (off, size) in enumerate(_CHUNKS):
        idx_copies[j].wait()
        gathers.append(
            pltpu.async_copy(
                x_hbm.at[idx_v.at[pl.ds(off, size)]],
                rows_v.at[pl.ds(off, size)],
                gather_sems[j],
            )
        )
    outs = []
    for j, (off, size) in enumerate(_CHUNKS):
        gathers[j].wait()
        outs.append(
            pltpu.async_copy(
                rows_v.at[pl.ds(off, size)],
                out_hbm.at[pl.ds(base + off, size)],
                out_sem,
            )
        )
    for cp in outs:
        cp.wait()


def kernel(node_indices, x, edge_index, edge_type, edge_attr):
    del edge_index, edge_type, edge_attr
    return _gather_rows(node_indices, x)
